# baseline (speedup 1.0000x reference)
import jax
import jax.numpy as jnp
from jax import lax
from jax.experimental import pallas as pl
from jax.experimental.pallas import tpu as pltpu
from jax.experimental.pallas import tpu_sc as plsc

BATCH = 16384
SEQ = 200
EMB = 64

NC = 2
NS = 16
NW = NC * NS

G = 128
K = 4
CHUNK = G * K

TOTAL = BATCH * SEQ
GROUPS = TOTAL // G
GROUPS_PER_W = GROUPS // NW
BODIES = GROUPS_PER_W // (2 * K)


def _body(idx_hbm, table_hbm, out_hbm, idx_v, rows0, rows1,
          sem_g0, sem_g1, sem_w0, sem_w1):
    wid = lax.axis_index("s") * NC + lax.axis_index("c")
    g_base = wid * GROUPS_PER_W
    rows = (rows0, rows1)
    sem_g = (sem_g0, sem_g1)
    sem_w = (sem_w0, sem_w1)

    def step(it, _):
        g0 = g_base + it * 2 * K
        pltpu.sync_copy(idx_hbm.at[pl.ds(g0, 2 * K)], idx_v)
        gather_handles = ([], [])
        for b in range(2):
            @pl.when(it > 0)
            def _drain():
                pltpu.make_async_copy(
                    rows[b], out_hbm.at[pl.ds(0, CHUNK)], sem_w[b]
                ).wait()
            for j in range(K):
                gather_handles[b].append(
                    pltpu.async_copy(
                        table_hbm.at[idx_v.at[b * K + j]],
                        rows[b].at[pl.ds(j * G, G)],
                        sem_g[b],
                    )
                )
        for b in range(2):
            for h in gather_handles[b]:
                h.wait()
            pltpu.async_copy(
                rows[b],
                out_hbm.at[pl.ds((g0 + b * K) * G, CHUNK)],
                sem_w[b],
            )
        return 0

    lax.fori_loop(0, BODIES, step, 0)
    for b in range(2):
        pltpu.make_async_copy(
            rows[b], out_hbm.at[pl.ds(0, CHUNK)], sem_w[b]
        ).wait()


def kernel(visit_segments, table):
    idx = visit_segments.reshape(GROUPS, G).astype(jnp.int32)
    mesh = plsc.VectorSubcoreMesh(
        core_axis_name="c", subcore_axis_name="s",
        num_cores=NC, num_subcores=NS,
    )
    grab = pl.kernel(
        _body,
        out_type=jax.ShapeDtypeStruct((TOTAL, EMB), jnp.float32),
        mesh=mesh,
        scratch_types=[
            pltpu.VMEM((2 * K, G), jnp.int32),
            pltpu.VMEM((CHUNK, EMB), jnp.float32),
            pltpu.VMEM((CHUNK, EMB), jnp.float32),
            pltpu.SemaphoreType.DMA,
            pltpu.SemaphoreType.DMA,
            pltpu.SemaphoreType.DMA,
            pltpu.SemaphoreType.DMA,
        ],
        compiler_params=pltpu.CompilerParams(use_tc_tiling_on_sc=False),
    )
    out = grab(idx, table)
    return out.reshape(BATCH, SEQ, EMB)


# speedup vs baseline: 1.1201x; 1.1201x over previous
import jax
import jax.numpy as jnp
from jax import lax
from jax.experimental import pallas as pl
from jax.experimental.pallas import tpu as pltpu
from jax.experimental.pallas import tpu_sc as plsc

BATCH = 16384
SEQ = 200
EMB = 64
VOCAB = 1000

NC = 2
NS = 16
NW = NC * NS

B_PER_W = BATCH // NW
CB = 256
HALVES = B_PER_W // CB
L = 16


def _body(idx_hbm, table_hbm, out_hbm, table_v, idx_v, blk0, blk1,
          sem_i, sem_w0, sem_w1):
    wid = lax.axis_index("s") * NC + lax.axis_index("c")
    b0 = wid * B_PER_W
    blk = (blk0, blk1)
    sem_w = (sem_w0, sem_w1)

    pltpu.sync_copy(table_hbm, table_v)
    pltpu.async_copy(idx_hbm.at[0, pl.ds(b0, B_PER_W)], idx_v.at[0], sem_i)

    def per_seq(s, _):
        par = s % 2
        pltpu.make_async_copy(
            idx_hbm.at[0, pl.ds(b0, B_PER_W)], idx_v.at[0], sem_i
        ).wait()

        @pl.when(s < SEQ - 1)
        def _prefetch():
            pltpu.async_copy(
                idx_hbm.at[s + 1, pl.ds(b0, B_PER_W)],
                idx_v.at[(s + 1) % 2], sem_i,
            )

        for h in range(HALVES):
            @pl.when(s > 0)
            def _drain():
                pltpu.make_async_copy(
                    blk[h], out_hbm.at[0, slice(None), pl.ds(0, CB)], sem_w[h]
                ).wait()
            for i in range(CB // L):
                col = idx_v[par, pl.ds(h * CB + i * L, L)]
                for e in range(EMB):
                    row = jnp.full((L,), e, jnp.int32)
                    blk[h][e, pl.ds(i * L, L)] = plsc.load_gather(
                        table_v, [row, col]
                    )
            pltpu.async_copy(
                blk[h], out_hbm.at[s, slice(None), pl.ds(b0 + h * CB, CB)],
                sem_w[h],
            )
        return 0

    lax.fori_loop(0, SEQ, per_seq, 0)
    for h in range(HALVES):
        pltpu.make_async_copy(
            blk[h], out_hbm.at[0, slice(None), pl.ds(0, CB)], sem_w[h]
        ).wait()


def kernel(visit_segments, table):
    idx_t = visit_segments.T
    table_t = table.T
    mesh = plsc.VectorSubcoreMesh(
        core_axis_name="c", subcore_axis_name="s",
        num_cores=NC, num_subcores=NS,
    )
    grab = pl.kernel(
        _body,
        out_type=jax.ShapeDtypeStruct((SEQ, EMB, BATCH), jnp.float32),
        mesh=mesh,
        scratch_types=[
            pltpu.VMEM((EMB, VOCAB), jnp.float32),
            pltpu.VMEM((2, B_PER_W), jnp.int32),
            pltpu.VMEM((EMB, CB), jnp.float32),
            pltpu.VMEM((EMB, CB), jnp.float32),
            pltpu.SemaphoreType.DMA,
            pltpu.SemaphoreType.DMA,
            pltpu.SemaphoreType.DMA,
        ],
        compiler_params=pltpu.CompilerParams(
            use_tc_tiling_on_sc=True, needs_layout_passes=False,
        ),
    )
    out_phys = grab(idx_t, table_t)
    return out_phys.transpose(2, 0, 1)


# speedup vs baseline: 5.3615x; 4.7867x over previous
import jax
import jax.numpy as jnp
from jax import lax
from jax.experimental import pallas as pl
from jax.experimental.pallas import tpu as pltpu
from jax.experimental.pallas import tpu_sc as plsc

BATCH = 16384
SEQ = 200
EMB = 64
VOCAB = 1000

NC = 2
NS = 16
NW = NC * NS

B_PER_W = BATCH // NW
CB = 256
HALVES = B_PER_W // CB
L = 16


def _body(idx_hbm, table_hbm, out_hbm, table_v, idx_v, blk0, blk1,
          sem_i, sem_w0, sem_w1):
    wid = lax.axis_index("s") * NC + lax.axis_index("c")
    b0 = wid * B_PER_W
    blk = (blk0, blk1)
    sem_w = (sem_w0, sem_w1)

    pltpu.sync_copy(table_hbm, table_v)
    pltpu.async_copy(idx_hbm.at[0, pl.ds(b0, B_PER_W)], idx_v.at[0], sem_i)

    def per_seq(s, _):
        par = s % 2
        pltpu.make_async_copy(
            idx_hbm.at[0, pl.ds(b0, B_PER_W)], idx_v.at[0], sem_i
        ).wait()

        @pl.when(s < SEQ - 1)
        def _prefetch():
            pltpu.async_copy(
                idx_hbm.at[s + 1, pl.ds(b0, B_PER_W)],
                idx_v.at[(s + 1) % 2], sem_i,
            )

        for h in range(HALVES):
            @pl.when(s > 0)
            def _drain():
                pltpu.make_async_copy(
                    blk[h], out_hbm.at[0, slice(None), pl.ds(0, CB)], sem_w[h]
                ).wait()
            @plsc.parallel_loop(0, CB // L, 1, unroll=4)
            def _gather(i):
                col = idx_v[par, pl.ds(h * CB + i * L, L)]
                for e in range(EMB):
                    row = jnp.full((L,), e, jnp.int32)
                    blk[h][e, pl.ds(i * L, L)] = plsc.load_gather(
                        table_v, [row, col]
                    )
            pltpu.async_copy(
                blk[h], out_hbm.at[s, slice(None), pl.ds(b0 + h * CB, CB)],
                sem_w[h],
            )
        return 0

    lax.fori_loop(0, SEQ, per_seq, 0)
    for h in range(HALVES):
        pltpu.make_async_copy(
            blk[h], out_hbm.at[0, slice(None), pl.ds(0, CB)], sem_w[h]
        ).wait()


def kernel(visit_segments, table):
    idx_t = visit_segments.T
    table_t = table.T
    mesh = plsc.VectorSubcoreMesh(
        core_axis_name="c", subcore_axis_name="s",
        num_cores=NC, num_subcores=NS,
    )
    grab = pl.kernel(
        _body,
        out_type=jax.ShapeDtypeStruct((SEQ, EMB, BATCH), jnp.float32),
        mesh=mesh,
        scratch_types=[
            pltpu.VMEM((EMB, VOCAB), jnp.float32),
            pltpu.VMEM((2, B_PER_W), jnp.int32),
            pltpu.VMEM((EMB, CB), jnp.float32),
            pltpu.VMEM((EMB, CB), jnp.float32),
            pltpu.SemaphoreType.DMA,
            pltpu.SemaphoreType.DMA,
            pltpu.SemaphoreType.DMA,
        ],
        compiler_params=pltpu.CompilerParams(
            use_tc_tiling_on_sc=True, needs_layout_passes=False,
        ),
    )
    out_phys = grab(idx_t, table_t)
    return out_phys.transpose(2, 0, 1)
